# baseline (speedup 1.0000x reference)
import functools

import jax
import jax.numpy as jnp
from jax import lax
from jax.experimental import pallas as pl
from jax.experimental.pallas import tpu as pltpu
from jax.experimental.pallas import tpu_sc as plsc

B, D, H, W = 16, 8, 56, 56
M = D * H * W
WA = H * W
HW = WA // 2
NVEC = HW // 16
PATHS = ((4, 2), (8, 3), (12, 4))
NPG = sum(p for p, _ in PATHS)
NOUT = NPG * WA
LMAX = 4

_PATH_OF = []
_row = 0
for _t, (_P, _L) in enumerate(PATHS):
    for _p in range(_P):
        _PATH_OF.append((_L, _row))
        _row += _L
NIROWS = _row


def _sc_body(cat_hbm, out_hbm,
             tab_v, idx_v, out_v, sem_t, sem_i0, sem_i1, sem_i2, sem_i3,
             sem_o):
    sem_i = (sem_i0, sem_i1, sem_i2, sem_i3)
    b = lax.axis_index("s")
    h = lax.axis_index("c")
    w0 = h * HW

    tab_h = pltpu.async_copy(cat_hbm.at[pl.ds(b * M, M)], tab_v, sem_t)

    def fire_idx(pg):
        L, row = _PATH_OF[pg]
        pk = pg % 4
        off = B * M + row * WA + w0
        hs = []
        for l in range(L):
            hs.append(pltpu.async_copy(
                cat_hbm.at[pl.ds(off + l * WA, HW)],
                idx_v.at[pk, l], sem_i[pk]))
        return hs

    def compute(pg):
        L, row = _PATH_OF[pg]
        pk = pg % 4

        @plsc.parallel_loop(0, HW, 16, unroll=7)
        def body(g):
            s = pl.ds(g, 16)
            v = plsc.bitcast(
                plsc.load_gather(tab_v, [idx_v[pk, 0, s]]), jnp.float32)
            for l in range(1, L):
                v = jnp.maximum(v, plsc.bitcast(
                    plsc.load_gather(tab_v, [idx_v[pk, l, s]]), jnp.float32))
            out_v[pk, s] = 1.0 - v

    def fire_out(pg):
        pk = pg % 4
        col = b * NOUT + pg * WA + w0
        return pltpu.async_copy(out_v.at[pk], out_hbm.at[pl.ds(col, HW)],
                                sem_o)

    idx_h = {pg: fire_idx(pg) for pg in range(3)}
    out_h = {}
    tab_waited = False
    for pg in range(NPG):
        if pg + 3 < NPG:
            idx_h[pg + 3] = fire_idx(pg + 3)
        for hnd in idx_h.pop(pg):
            hnd.wait()
        if not tab_waited:
            tab_h.wait()
            tab_waited = True
        if pg - 4 in out_h:
            out_h.pop(pg - 4).wait()
        compute(pg)
        out_h[pg] = fire_out(pg)
    for hnd in out_h.values():
        hnd.wait()


@jax.jit
def _sc_call(cat):
    mesh = plsc.VectorSubcoreMesh(core_axis_name="c", subcore_axis_name="s")
    return pl.kernel(
        _sc_body,
        out_type=jax.ShapeDtypeStruct((B * NOUT,), jnp.float32),
        mesh=mesh,
        scratch_types=[
            pltpu.VMEM((M,), jnp.int32),
            pltpu.VMEM((4, LMAX, HW), jnp.int32),
            pltpu.VMEM((4, HW), jnp.float32),
            pltpu.SemaphoreType.DMA,
            pltpu.SemaphoreType.DMA,
            pltpu.SemaphoreType.DMA,
            pltpu.SemaphoreType.DMA,
            pltpu.SemaphoreType.DMA,
            pltpu.SemaphoreType.DMA,
        ],
        compiler_params=pltpu.CompilerParams(
            use_tc_tiling_on_sc=False, needs_layout_passes=False),
    )(cat)


def kernel(x, path_indices_0, path_indices_1, path_indices_2):
    cat = jnp.concatenate(
        [jax.lax.bitcast_convert_type(x, jnp.int32).reshape(-1),
         path_indices_0.reshape(-1),
         path_indices_1.reshape(-1),
         path_indices_2.reshape(-1)])
    out = _sc_call(cat)
    return out.reshape(B, NPG, WA)


# speedup vs baseline: 1.0560x; 1.0560x over previous
import functools

import jax
import jax.numpy as jnp
from jax import lax
from jax.experimental import pallas as pl
from jax.experimental.pallas import tpu as pltpu
from jax.experimental.pallas import tpu_sc as plsc

B, D, H, W = 16, 8, 56, 56
M = D * H * W
WA = H * W
HW = WA // 2
NVEC = HW // 16
PATHS = ((4, 2), (8, 3), (12, 4))
NPG = sum(p for p, _ in PATHS)
NOUT = NPG * WA
LMAX = 4

_PATH_OF = []
_row = 0
for _t, (_P, _L) in enumerate(PATHS):
    for _p in range(_P):
        _PATH_OF.append((_L, _row))
        _row += _L
NIROWS = _row


def _sc_body(x_hbm, idx_hbm, out_hbm,
             tab_v, idx_v, out_v, sem_t, sem_i0, sem_i1, sem_i2, sem_i3,
             sem_o):
    sem_i = (sem_i0, sem_i1, sem_i2, sem_i3)
    b = lax.axis_index("s")
    h = lax.axis_index("c")
    w0 = h * HW

    tab_h = pltpu.async_copy(x_hbm.at[pl.ds(b * M, M)], tab_v, sem_t)

    def fire_idx(pg):
        L, row = _PATH_OF[pg]
        pk = pg % 4
        return pltpu.async_copy(
            idx_hbm.at[pl.ds(row, L), pl.ds(w0, HW)],
            idx_v.at[pk, pl.ds(0, L)], sem_i[pk])

    def compute(pg):
        L, row = _PATH_OF[pg]
        pk = pg % 4

        @plsc.parallel_loop(0, HW, 16, unroll=7)
        def body(g):
            s = pl.ds(g, 16)
            v = plsc.load_gather(tab_v, [idx_v[pk, 0, s]])
            for l in range(1, L):
                v = jnp.maximum(v, plsc.load_gather(tab_v, [idx_v[pk, l, s]]))
            out_v[pk, s] = 1.0 - v

    def fire_out(pg):
        pk = pg % 4
        return pltpu.async_copy(out_v.at[pk],
                                out_hbm.at[b, pg, pl.ds(w0, HW)], sem_o)

    idx_h = {pg: fire_idx(pg) for pg in range(3)}
    out_h = {}
    tab_waited = False
    for pg in range(NPG):
        if pg + 3 < NPG:
            idx_h[pg + 3] = fire_idx(pg + 3)
        idx_h.pop(pg).wait()
        if not tab_waited:
            tab_h.wait()
            tab_waited = True
        if pg - 4 in out_h:
            out_h.pop(pg - 4).wait()
        compute(pg)
        out_h[pg] = fire_out(pg)
    for hnd in out_h.values():
        hnd.wait()


@jax.jit
def _sc_call(x_flat, idx_all):
    mesh = plsc.VectorSubcoreMesh(core_axis_name="c", subcore_axis_name="s")
    return pl.kernel(
        _sc_body,
        out_type=jax.ShapeDtypeStruct((B, NPG, WA), jnp.float32),
        mesh=mesh,
        scratch_types=[
            pltpu.VMEM((M,), jnp.float32),
            pltpu.VMEM((4, LMAX, HW), jnp.int32),
            pltpu.VMEM((4, HW), jnp.float32),
            pltpu.SemaphoreType.DMA,
            pltpu.SemaphoreType.DMA,
            pltpu.SemaphoreType.DMA,
            pltpu.SemaphoreType.DMA,
            pltpu.SemaphoreType.DMA,
            pltpu.SemaphoreType.DMA,
        ],
        compiler_params=pltpu.CompilerParams(
            use_tc_tiling_on_sc=False, needs_layout_passes=False),
    )(x_flat, idx_all)


def kernel(x, path_indices_0, path_indices_1, path_indices_2):
    idx_all = jnp.concatenate(
        [path_indices_0.reshape(-1, WA),
         path_indices_1.reshape(-1, WA),
         path_indices_2.reshape(-1, WA)], axis=0)
    return _sc_call(x.reshape(-1), idx_all)
